# baseline (speedup 1.0000x reference)
import functools
import math

import jax
import jax.numpy as jnp
from jax import lax
from jax.experimental import pallas as pl
from jax.experimental.pallas import tpu as pltpu
from jax.experimental.pallas import tpu_sc as plsc

_LANES = 16
_BLK = 128


def _build_table_transpose(vocab, d_model, nc, ns):
    nw = nc * ns
    n_blocks = vocab // _BLK
    rem = vocab - n_blocks * _BLK
    nb = n_blocks + (1 if rem else 0)
    scale = math.sqrt(d_model)

    mesh = plsc.VectorSubcoreMesh(core_axis_name="c", subcore_axis_name="s")

    @functools.partial(
        pl.kernel,
        mesh=mesh,
        compiler_params=pltpu.CompilerParams(
            use_tc_tiling_on_sc=True, needs_layout_passes=False
        ),
        out_type=jax.ShapeDtypeStruct((nb, 8, 8, _BLK), jnp.float32),
        scratch_types=[
            pltpu.VMEM((2, d_model, _BLK), jnp.float32),
            pltpu.VMEM((2, 8, 8, _BLK), jnp.float32),
            pltpu.SemaphoreType.DMA,
            pltpu.SemaphoreType.DMA,
            pltpu.SemaphoreType.DMA,
            pltpu.SemaphoreType.DMA,
        ],
    )
    def run(lutt_hbm, tail_hbm, out_hbm, gbuf, tbuf, gs0, gs1, os0, os1):
        gsems = [gs0, gs1]
        osems = [os0, os1]
        wid = lax.axis_index("s") * nc + lax.axis_index("c")
        my_n = lax.select(
            wid < n_blocks - (n_blocks // nw) * nw,
            (n_blocks // nw) + 1,
            n_blocks // nw,
        )

        def blk_id(k):
            return k * nw + wid

        def gather(k, b):
            c = blk_id(k)
            pltpu.async_copy(
                lutt_hbm.at[:, pl.ds(c * _BLK, _BLK)], gbuf.at[b], gsems[b]
            )

        def gwait(k, b):
            c = blk_id(k)
            pltpu.make_async_copy(
                lutt_hbm.at[:, pl.ds(c * _BLK, _BLK)], gbuf.at[b], gsems[b]
            ).wait()

        def scatter(k, b):
            pltpu.async_copy(tbuf.at[b], out_hbm.at[blk_id(k)], osems[b])

        def owait(k, b):
            pltpu.make_async_copy(
                tbuf.at[b], out_hbm.at[blk_id(k)], osems[b]
            ).wait()

        lane_iota = lax.iota(jnp.int32, _LANES)
        mask = d_model - 1

        def transpose_blk(b):
            for j in range(_BLK // _LANES):
                rows = lane_iota + (j * _LANES)

                @pl.loop(0, d_model, unroll=4)
                def m_body(m):
                    perm = (lane_iota + m) & mask
                    flat = rows * d_model + perm
                    a = lax.shift_right_logical(flat, 10)
                    bb_ = lax.shift_right_logical(flat, 7) & 7
                    cc = flat & (_BLK - 1)
                    vals = plsc.load_gather(gbuf.at[b], [perm, rows])
                    plsc.store_scatter(tbuf.at[b], [a, bb_, cc], vals * scale)

        @pl.when(my_n > 0)
        def _():
            gather(0, 0)

            @pl.when(my_n > 1)
            def _():
                gather(1, 1)

            def step(k, carry):
                def do(bb):
                    gwait(k, bb)

                    @pl.when(k >= 2)
                    def _():
                        owait(k - 2, bb)

                    transpose_blk(bb)

                    @pl.when(k + 2 < my_n)
                    def _():
                        gather(k + 2, bb)

                    scatter(k, bb)

                lax.cond(lax.rem(k, 2) == 0, lambda: do(0), lambda: do(1))
                return carry

            lax.fori_loop(0, my_n, step, 0)

            def drain(k):
                lax.cond(
                    lax.rem(k, 2) == 0, lambda: owait(k, 0), lambda: owait(k, 1)
                )

            @pl.when(my_n > 1)
            def _():
                drain(my_n - 2)

            drain(my_n - 1)

        if rem:

            @pl.when(wid == 0)
            def _():
                pltpu.sync_copy(tail_hbm, gbuf.at[0])
                transpose_blk(0)
                pltpu.sync_copy(tbuf.at[0], out_hbm.at[n_blocks])

    return run, nb


def _build_lookup(n_tok, d_model, nrows, nc, ns):
    nw = nc * ns
    n_trow = n_tok // 8

    mesh = plsc.VectorSubcoreMesh(core_axis_name="c", subcore_axis_name="s")

    @functools.partial(
        pl.kernel,
        mesh=mesh,
        compiler_params=pltpu.CompilerParams(
            use_tc_tiling_on_sc=False, needs_layout_passes=False
        ),
        out_type=jax.ShapeDtypeStruct((n_tok, 8, nw, 8, _BLK), jnp.float32),
        scratch_types=[
            pltpu.VMEM((n_trow, 8, _BLK), jnp.int32),
            pltpu.VMEM((3, _BLK, d_model), jnp.float32),
            pltpu.VMEM((3, 8, 8, _BLK), jnp.float32),
            pltpu.SemaphoreType.DMA,
            pltpu.SemaphoreType.DMA,
            pltpu.SemaphoreType.DMA,
            pltpu.SemaphoreType.DMA,
            pltpu.SemaphoreType.DMA,
            pltpu.SemaphoreType.DMA,
        ],
    )
    def run(
        x_hbm, lut_hbm, out_hbm, idx_v, gbuf, obuf, gs0, gs1, gs2, os0, os1, os2
    ):
        gsems = [gs0, gs1, gs2]
        osems = [os0, os1, os2]
        wid = lax.axis_index("s") * nc + lax.axis_index("c")
        pltpu.sync_copy(x_hbm.at[:, wid], idx_v)

        def idx_row(c):
            return idx_v.at[lax.shift_right_logical(c, 3), lax.rem(c, 8)]

        def gather(c, b):
            pltpu.async_copy(lut_hbm.at[idx_row(c)], gbuf.at[b], gsems[b])

        def gwait(c, b):
            pltpu.make_async_copy(
                lut_hbm.at[idx_row(c)], gbuf.at[b], gsems[b]
            ).wait()

        def out_slice(c):
            return out_hbm.at[c, :, wid]

        def scatter(c, b):
            pltpu.async_copy(obuf.at[b], out_slice(c), osems[b])

        def owait(c, b):
            pltpu.make_async_copy(obuf.at[b], out_slice(c), osems[b]).wait()

        lane_iota = lax.iota(jnp.int32, _LANES)
        mask = d_model - 1

        def transpose_blk(b):
            @pl.loop(0, d_model, unroll=4)
            def m_body(m):
                perm = (lane_iota + m) & mask
                pi = lax.shift_right_logical(perm, 3)
                ps = perm & 7
                for j in range(_BLK // _LANES):
                    rows = lane_iota + (j * _LANES)
                    vals = plsc.load_gather(gbuf.at[b], [rows, perm])
                    plsc.store_scatter(obuf.at[b], [pi, ps, rows], vals)

        n_chunks = n_tok
        n_full = n_chunks // 3

        for b in range(3):
            gather(b, b)

        for b in range(3):
            gwait(b, b)
            transpose_blk(b)
            gather(b + 3, b)
            scatter(b, b)

        def group_body(g, carry):
            for b in range(3):
                c = g * 3 + b
                gwait(c, b)
                owait(c - 3, b)
                transpose_blk(b)

                @pl.when(c + 3 < n_chunks)
                def _():
                    gather(c + 3, b)

                scatter(c, b)
            return carry

        lax.fori_loop(1, n_full, group_body, 0)

        for b in range(n_chunks - 3 * n_full):
            c = 3 * n_full + b
            gwait(c, b)
            owait(c - 3, b)
            transpose_blk(b)
            scatter(c, b)

        for c in range(n_chunks - 3, n_chunks):
            owait(c, c % 3)

    return run


def kernel(x, lut):
    bsz, n_tok = x.shape
    vocab, d_model = lut.shape

    info = plsc.get_sparse_core_info()
    nc, ns = info.num_cores, info.num_subcores
    nw = nc * ns

    x4 = (
        x.T.astype(jnp.int32)
        .reshape(n_tok // 8, 8, nw, _BLK)
        .transpose(0, 2, 1, 3)
    )
    t_run, nb = _build_table_transpose(vocab, d_model, nc, ns)
    rem = vocab % _BLK
    tail_rows = _BLK if rem else 0
    tail = jnp.pad(
        lax.slice(lut, (vocab - rem, 0), (vocab, d_model)),
        ((0, _BLK - rem), (0, 0)),
    ).T
    scratch4 = t_run(lut.T, tail)
    lut_rows = scratch4.reshape(nb * _BLK, d_model)

    run = _build_lookup(n_tok, d_model, nb * _BLK, nc, ns)
    out5 = run(x4, lut_rows)
    out = out5.transpose(2, 4, 0, 1, 3).reshape(bsz, n_tok, d_model)
    return out


# speedup vs baseline: 1.0082x; 1.0082x over previous
import functools
import math

import jax
import jax.numpy as jnp
from jax import lax
from jax.experimental import pallas as pl
from jax.experimental.pallas import tpu as pltpu
from jax.experimental.pallas import tpu_sc as plsc

_LANES = 16
_BLK = 128


def _build_table_transpose(vocab, d_model, nc, ns):
    nw = nc * ns
    n_blocks = vocab // _BLK
    rem = vocab - n_blocks * _BLK
    nb = n_blocks + (1 if rem else 0)
    scale = math.sqrt(d_model)

    mesh = plsc.VectorSubcoreMesh(core_axis_name="c", subcore_axis_name="s")

    @functools.partial(
        pl.kernel,
        mesh=mesh,
        compiler_params=pltpu.CompilerParams(
            use_tc_tiling_on_sc=True, needs_layout_passes=False
        ),
        out_type=jax.ShapeDtypeStruct((nb, 8, 8, _BLK), jnp.float32),
        scratch_types=[
            pltpu.VMEM((2, d_model, _BLK), jnp.float32),
            pltpu.VMEM((2, 8, 8, _BLK), jnp.float32),
            pltpu.SemaphoreType.DMA,
            pltpu.SemaphoreType.DMA,
            pltpu.SemaphoreType.DMA,
            pltpu.SemaphoreType.DMA,
        ],
    )
    def run(lutt_hbm, tail_hbm, out_hbm, gbuf, tbuf, gs0, gs1, os0, os1):
        gsems = [gs0, gs1]
        osems = [os0, os1]
        wid = lax.axis_index("s") * nc + lax.axis_index("c")
        my_n = lax.select(
            wid < n_blocks - (n_blocks // nw) * nw,
            (n_blocks // nw) + 1,
            n_blocks // nw,
        )

        def blk_id(k):
            return k * nw + wid

        def gather(k, b):
            c = blk_id(k)
            pltpu.async_copy(
                lutt_hbm.at[:, pl.ds(c * _BLK, _BLK)], gbuf.at[b], gsems[b]
            )

        def gwait(k, b):
            c = blk_id(k)
            pltpu.make_async_copy(
                lutt_hbm.at[:, pl.ds(c * _BLK, _BLK)], gbuf.at[b], gsems[b]
            ).wait()

        def scatter(k, b):
            pltpu.async_copy(tbuf.at[b], out_hbm.at[blk_id(k)], osems[b])

        def owait(k, b):
            pltpu.make_async_copy(
                tbuf.at[b], out_hbm.at[blk_id(k)], osems[b]
            ).wait()

        lane_iota = lax.iota(jnp.int32, _LANES)
        mask = d_model - 1

        def transpose_blk(b):
            for j in range(_BLK // _LANES):
                rows = lane_iota + (j * _LANES)

                @pl.loop(0, d_model, unroll=4)
                def m_body(m):
                    perm = (lane_iota + m) & mask
                    flat = rows * d_model + perm
                    a = lax.shift_right_logical(flat, 10)
                    bb_ = lax.shift_right_logical(flat, 7) & 7
                    cc = flat & (_BLK - 1)
                    vals = plsc.load_gather(gbuf.at[b], [perm, rows])
                    plsc.store_scatter(tbuf.at[b], [a, bb_, cc], vals * scale)

        @pl.when(my_n > 0)
        def _():
            gather(0, 0)

            @pl.when(my_n > 1)
            def _():
                gather(1, 1)

            def step(k, carry):
                def do(bb):
                    gwait(k, bb)

                    @pl.when(k >= 2)
                    def _():
                        owait(k - 2, bb)

                    transpose_blk(bb)

                    @pl.when(k + 2 < my_n)
                    def _():
                        gather(k + 2, bb)

                    scatter(k, bb)

                lax.cond(lax.rem(k, 2) == 0, lambda: do(0), lambda: do(1))
                return carry

            lax.fori_loop(0, my_n, step, 0)

            def drain(k):
                lax.cond(
                    lax.rem(k, 2) == 0, lambda: owait(k, 0), lambda: owait(k, 1)
                )

            @pl.when(my_n > 1)
            def _():
                drain(my_n - 2)

            drain(my_n - 1)

        if rem:

            @pl.when(wid == 0)
            def _():
                pltpu.sync_copy(tail_hbm, gbuf.at[0])
                transpose_blk(0)
                pltpu.sync_copy(tbuf.at[0], out_hbm.at[n_blocks])

    return run, nb


def _build_lookup(n_tok, d_model, nrows, nc, ns):
    nw = nc * ns
    n_trow = n_tok // 8

    mesh = plsc.VectorSubcoreMesh(core_axis_name="c", subcore_axis_name="s")

    @functools.partial(
        pl.kernel,
        mesh=mesh,
        compiler_params=pltpu.CompilerParams(
            use_tc_tiling_on_sc=False, needs_layout_passes=False
        ),
        out_type=jax.ShapeDtypeStruct((n_tok, 8, nw, 8, _BLK), jnp.float32),
        scratch_types=[
            pltpu.VMEM((n_trow, 8, _BLK), jnp.int32),
            pltpu.VMEM((2, _BLK, d_model), jnp.float32),
            pltpu.VMEM((2, 8, 8, _BLK), jnp.float32),
            pltpu.SemaphoreType.DMA,
            pltpu.SemaphoreType.DMA,
            pltpu.SemaphoreType.DMA,
            pltpu.SemaphoreType.DMA,
        ],
    )
    def run(x_hbm, lut_hbm, out_hbm, idx_v, gbuf, obuf, gs0, gs1, os0, os1):
        gsems = [gs0, gs1]
        osems = [os0, os1]
        wid = lax.axis_index("s") * nc + lax.axis_index("c")
        pltpu.sync_copy(x_hbm.at[:, wid], idx_v)

        def idx_row(c):
            return idx_v.at[lax.shift_right_logical(c, 3), lax.rem(c, 8)]

        def gather(c, b):
            pltpu.async_copy(lut_hbm.at[idx_row(c)], gbuf.at[b], gsems[b])

        def gwait(c, b):
            pltpu.make_async_copy(
                lut_hbm.at[idx_row(c)], gbuf.at[b], gsems[b]
            ).wait()

        def out_slice(c):
            return out_hbm.at[c, :, wid]

        def scatter(c, b):
            pltpu.async_copy(obuf.at[b], out_slice(c), osems[b])

        def owait(c, b):
            pltpu.make_async_copy(obuf.at[b], out_slice(c), osems[b]).wait()

        lane_iota = lax.iota(jnp.int32, _LANES)
        mask = d_model - 1

        def transpose_blk(b):
            @pl.loop(0, d_model, unroll=8)
            def m_body(m):
                perm = (lane_iota + m) & mask
                pi = lax.shift_right_logical(perm, 3)
                ps = perm & 7
                for j in range(_BLK // _LANES):
                    rows = lane_iota + (j * _LANES)
                    vals = plsc.load_gather(gbuf.at[b], [rows, perm])
                    plsc.store_scatter(obuf.at[b], [pi, ps, rows], vals)

        n_chunks = n_tok

        for b in range(2):
            gather(b, b)

        for b in range(2):
            gwait(b, b)
            transpose_blk(b)
            gather(b + 2, b)
            scatter(b, b)

        def group_body(g, carry):
            for b in range(2):
                c = g * 2 + b
                gwait(c, b)
                owait(c - 2, b)
                transpose_blk(b)
                gather(c + 2, b)
                scatter(c, b)
            return carry

        lax.fori_loop(1, n_chunks // 2 - 1, group_body, 0)

        for b in range(2):
            c = n_chunks - 2 + b
            gwait(c, b)
            owait(c - 2, b)
            transpose_blk(b)
            scatter(c, b)

        for b in range(2):
            owait(n_chunks - 2 + b, b)

    return run


def kernel(x, lut):
    bsz, n_tok = x.shape
    vocab, d_model = lut.shape

    info = plsc.get_sparse_core_info()
    nc, ns = info.num_cores, info.num_subcores
    nw = nc * ns

    x4 = (
        x.T.astype(jnp.int32)
        .reshape(n_tok // 8, 8, nw, _BLK)
        .transpose(0, 2, 1, 3)
    )
    t_run, nb = _build_table_transpose(vocab, d_model, nc, ns)
    rem = vocab % _BLK
    tail_rows = _BLK if rem else 0
    tail = jnp.pad(
        lax.slice(lut, (vocab - rem, 0), (vocab, d_model)),
        ((0, _BLK - rem), (0, 0)),
    ).T
    scratch4 = t_run(lut.T, tail)
    lut_rows = scratch4.reshape(nb * _BLK, d_model)

    run = _build_lookup(n_tok, d_model, nb * _BLK, nc, ns)
    out5 = run(x4, lut_rows)
    out = out5.transpose(2, 4, 0, 1, 3).reshape(bsz, n_tok, d_model)
    return out
